# baseline (speedup 1.0000x reference)
import functools

import jax
import jax.numpy as jnp
from jax import lax
from jax.experimental import pallas as pl
from jax.experimental.pallas import tpu as pltpu
from jax.experimental.pallas import tpu_sc as plsc

B = 4096
N_ACT = 131072
N_KEYS = 262144
E_VOCAB = 100000
R_VOCAB = 1000
ENT_DIM = 200
REL_DIM = 200
HIST_DIM = 200
NPAIR = ENT_DIM // 2
PK = 128

NC = 2
NS = 16
NW = NC * NS
L = 16


def _mesh():
    return plsc.VectorSubcoreMesh(
        core_axis_name="c", subcore_axis_name="s",
        num_cores=NC, num_subcores=NS)


def _wid():
    return lax.axis_index("s") * NC + lax.axis_index("c")


def _pack(table):
    tu = jax.lax.bitcast_convert_type(table, jnp.uint32)
    lo = (tu[:, 0::2] + 0x8000) >> 16
    hi = (tu[:, 1::2] + 0x8000) & jnp.uint32(0xFFFF0000)
    pk = jax.lax.bitcast_convert_type(hi | lo, jnp.int32)
    return jnp.pad(pk, ((0, 0), (0, PK - NPAIR)))


def _unpack_cols(pk, n_rows):
    t = jax.lax.bitcast_convert_type(pk[:, :NPAIR], jnp.bfloat16)
    return t.reshape(n_rows, ENT_DIM).astype(jnp.float32)


ROWS_PER_W = B // NW


@functools.partial(
    pl.kernel,
    out_type=(jax.ShapeDtypeStruct((B, PK), jnp.int32),
              jax.ShapeDtypeStruct((B, PK), jnp.int32)),
    mesh=_mesh(),
    scratch_types=[
        pltpu.VMEM((ROWS_PER_W,), jnp.int32),
        pltpu.VMEM((ROWS_PER_W,), jnp.int32),
        pltpu.VMEM((ROWS_PER_W, PK), jnp.int32),
        pltpu.VMEM((ROWS_PER_W, PK), jnp.int32),
        pltpu.SemaphoreType.DMA,
    ],
    compiler_params=pltpu.CompilerParams(use_tc_tiling_on_sc=True),
)
def _eq_gather(ent_hbm, rel_hbm, e_hbm, q_hbm, eout, qout,
               ei_v, qi_v, e_v, q_v, sem):
    base = _wid() * ROWS_PER_W
    pltpu.sync_copy(e_hbm.at[pl.ds(base, ROWS_PER_W)], ei_v)
    pltpu.sync_copy(q_hbm.at[pl.ds(base, ROWS_PER_W)], qi_v)
    c1 = pltpu.async_copy(ent_hbm.at[ei_v], e_v, sem)
    c2 = pltpu.async_copy(rel_hbm.at[qi_v], q_v, sem)
    c1.wait()
    c2.wait()
    pltpu.sync_copy(e_v, eout.at[pl.ds(base, ROWS_PER_W)])
    pltpu.sync_copy(q_v, qout.at[pl.ds(base, ROWS_PER_W)])


BLK = 256


def _mlp_body(e_ref, h_ref, q_ref, w1_ref, b1_ref, w2a_ref, w2b_ref,
              b2a_ref, b2b_ref, rel_ref, rp_ref, u_ref):
    x = jnp.dot(e_ref[...], w1_ref[0:ENT_DIM, :],
                preferred_element_type=jnp.float32)
    x = x + jnp.dot(h_ref[...], w1_ref[ENT_DIM:ENT_DIM + HIST_DIM, :],
                    preferred_element_type=jnp.float32)
    x = x + jnp.dot(q_ref[...], w1_ref[ENT_DIM + HIST_DIM:, :],
                    preferred_element_type=jnp.float32)
    x = jnp.maximum(x + b1_ref[...], 0.0)
    x2a = jnp.dot(x, w2a_ref[...], preferred_element_type=jnp.float32)
    x2a = x2a + b2a_ref[...]
    x2b = jnp.dot(x, w2b_ref[...], preferred_element_type=jnp.float32)
    x2b = x2b + b2b_ref[...]
    rp_ref[...] = lax.dot_general(
        x2a, rel_ref[...], (((1,), (1,)), ((), ())),
        preferred_element_type=jnp.float32)
    u_ref[...] = x2b


def _mlp(eemb, h, qemb, W1, b1, W2, b2, rel_table):
    W2a, W2b = W2[:, :REL_DIM], W2[:, REL_DIM:]
    b2a, b2b = b2[:REL_DIM].reshape(1, -1), b2[REL_DIM:].reshape(1, -1)
    return pl.pallas_call(
        _mlp_body,
        grid=(B // BLK,),
        in_specs=[
            pl.BlockSpec((BLK, ENT_DIM), lambda i: (i, 0)),
            pl.BlockSpec((BLK, HIST_DIM), lambda i: (i, 0)),
            pl.BlockSpec((BLK, REL_DIM), lambda i: (i, 0)),
            pl.BlockSpec((ENT_DIM + HIST_DIM + REL_DIM, 400), lambda i: (0, 0)),
            pl.BlockSpec((1, 400), lambda i: (0, 0)),
            pl.BlockSpec((400, REL_DIM), lambda i: (0, 0)),
            pl.BlockSpec((400, ENT_DIM), lambda i: (0, 0)),
            pl.BlockSpec((1, REL_DIM), lambda i: (0, 0)),
            pl.BlockSpec((1, ENT_DIM), lambda i: (0, 0)),
            pl.BlockSpec((R_VOCAB, REL_DIM), lambda i: (0, 0)),
        ],
        out_specs=[
            pl.BlockSpec((BLK, R_VOCAB), lambda i: (i, 0)),
            pl.BlockSpec((BLK, ENT_DIM), lambda i: (i, 0)),
        ],
        out_shape=[
            jax.ShapeDtypeStruct((B, R_VOCAB), jnp.float32),
            jax.ShapeDtypeStruct((B, ENT_DIM), jnp.float32),
        ],
    )(eemb, h, qemb, W1, b1.reshape(1, -1), W2a, W2b, b2a, b2b, rel_table)


KPW = N_KEYS // NW
NB = 128
NBLK = KPW // NB
NG = NB // L
MAXR = 32
ROT = 9
UNR = 10


@functools.partial(
    pl.kernel,
    out_type=jax.ShapeDtypeStruct((N_KEYS,), jnp.float32),
    mesh=_mesh(),
    scratch_types=[
        pltpu.VMEM((KPW,), jnp.int32),
        pltpu.VMEM((KPW,), jnp.int32),
        pltpu.VMEM((NB,), jnp.int32),
        pltpu.VMEM((NB,), jnp.int32),
        pltpu.VMEM((NB,), jnp.int32),
        pltpu.VMEM((NB,), jnp.float32),
        pltpu.VMEM((NB,), jnp.float32),
        pltpu.VMEM((NB, PK), jnp.int32),
        pltpu.VMEM((NB, PK), jnp.int32),
        pltpu.VMEM((NB, PK), jnp.int32),
        pltpu.VMEM((NB, PK), jnp.int32),
        pltpu.VMEM((NB,), jnp.float32),
        pltpu.VMEM((NB,), jnp.float32),
        pltpu.SemaphoreType.DMA,
        pltpu.SemaphoreType.DMA,
        pltpu.SemaphoreType.DMA,
    ],
    compiler_params=pltpu.CompilerParams(use_tc_tiling_on_sc=True, needs_layout_passes=False),
)
def _score(rp_hbm, u_hbm, ent_hbm, rs_hbm, es_hbm, ak_hbm, seg_hbm, out_hbm,
           ak_v, seg_v, rs_v, es_v, rpi_v, rpA, rpB, entA, entB, uA, uB,
           scA, scB, sem2, sem3, semo):
    base0 = _wid() * KPW

    def issue_s2(bi):
        idx = ak_v.at[pl.ds(bi * NB, NB)]
        pltpu.async_copy(rs_hbm.at[idx], rs_v, sem2)
        pltpu.async_copy(es_hbm.at[idx], es_v, sem2)

    def wait_s2():
        pltpu.make_async_copy(rs_hbm.at[ak_v.at[pl.ds(0, NB)]], rs_v, sem2).wait()
        pltpu.make_async_copy(es_hbm.at[ak_v.at[pl.ds(0, NB)]], es_v, sem2).wait()

    def block_meta(bi):
        seg0 = seg_v[pl.ds(bi * NB, L)][0]
        segl = seg_v[pl.ds(bi * NB + NB - L, L)][L - 1]
        base_u = jnp.minimum(seg0 & ~7, B - MAXR)
        is_lin = (segl - base_u) < MAXR
        return is_lin, base_u

    def issue_s3(bi, rp_b, ent_b, u_b):
        for g in range(NG):
            s16 = seg_v[pl.ds(bi * NB + g * L, L)]
            r16 = rs_v[pl.ds(g * L, L)]
            rpi_v[pl.ds(g * L, L)] = s16 * R_VOCAB + r16
        pltpu.async_copy(rp_hbm.at[rpi_v], rp_b, sem3)
        pltpu.async_copy(ent_hbm.at[es_v], ent_b, sem3)
        is_lin, base_u = block_meta(bi)

        @pl.when(is_lin)
        def _():
            pltpu.async_copy(u_hbm.at[pl.ds(pl.multiple_of(base_u, 8), MAXR)],
                             u_b.at[pl.ds(0, MAXR)], sem3)

        @pl.when(jnp.logical_not(is_lin))
        def _():
            pltpu.async_copy(u_hbm.at[seg_v.at[pl.ds(bi * NB, NB)]], u_b, sem3)

    def wait_s3(bi, rp_b, ent_b, u_b):
        pltpu.make_async_copy(rp_hbm.at[rpi_v], rp_b, sem3).wait()
        pltpu.make_async_copy(ent_hbm.at[es_v], ent_b, sem3).wait()
        is_lin, base_u = block_meta(bi)

        @pl.when(is_lin)
        def _():
            pltpu.make_async_copy(u_hbm.at[pl.ds(pl.multiple_of(base_u, 8), MAXR)],
                                  u_b.at[pl.ds(0, MAXR)], sem3).wait()

        @pl.when(jnp.logical_not(is_lin))
        def _():
            pltpu.make_async_copy(u_hbm.at[seg_v.at[pl.ds(0, NB)]],
                                  u_b, sem3).wait()

    def compute(bi, rp_b, ent_b, u_b, sc_b):
        rows = [lax.iota(jnp.int32, L) + g * L for g in range(NG)]
        zero = jnp.zeros((L,), jnp.float32)
        is_lin, base_u = block_meta(bi)
        m = is_lin.astype(jnp.int32)
        urows = []
        for g in range(NG):
            s16 = seg_v[pl.ds(bi * NB + g * L, L)]
            urows.append(m * (s16 - base_u) + (1 - m) * rows[g])

        col0 = (lax.iota(jnp.int32, L) * ROT) % NPAIR

        def dstep(j, accs):
            base_c = col0 + j * UNR
            cols = []
            for t in range(UNR):
                c = base_c + t
                cols.append(jnp.where(c >= NPAIR, c - NPAIR, c))
            out = []
            for g in range(NG):
                a = accs[g]
                for t in range(UNR):
                    epk = plsc.load_gather(ent_b, [rows[g], cols[t]])
                    upk = plsc.load_gather(u_b, [urows[g], cols[t]])
                    e0, e1 = plsc.unpack(plsc.bitcast(epk, jnp.bfloat16),
                                         format=plsc.PackFormat.INTERLEAVED)
                    u0, u1 = plsc.unpack(plsc.bitcast(upk, jnp.bfloat16),
                                         format=plsc.PackFormat.INTERLEAVED)
                    a = a + e0 * u0 + e1 * u1
                out.append(a)
            return tuple(out)

        accs = lax.fori_loop(0, NPAIR // UNR, dstep, (zero,) * NG)
        for g in range(NG):
            sc_b[pl.ds(g * L, L)] = accs[g] + rp_b[pl.ds(g * L, L)]

    def step(bi, cur, nxt):
        rp_c, ent_c, u_c, sc_c = cur
        rp_n, ent_n, u_n, _ = nxt
        base = base0 + bi * NB
        bnext = lax.min(bi + 1, NBLK - 1)
        wait_s3(bi, rp_c, ent_c, u_c)
        issue_s2(bnext)
        @pl.when(bi >= 2)
        def _():
            pltpu.make_async_copy(sc_c, out_hbm.at[pl.ds(base, NB)], semo).wait()
        compute(bi, rp_c, ent_c, u_c, sc_c)
        pltpu.async_copy(sc_c, out_hbm.at[pl.ds(base, NB)], semo)
        wait_s2()
        issue_s3(bnext, rp_n, ent_n, u_n)

    bufA = (rpA, entA, uA, scA)
    bufB = (rpB, entB, uB, scB)

    pltpu.sync_copy(ak_hbm.at[pl.ds(base0, KPW)], ak_v)
    pltpu.sync_copy(seg_hbm.at[pl.ds(base0, KPW)], seg_v)
    issue_s2(0)
    wait_s2()
    issue_s3(0, rpA, entA, uA)

    def pair(k, carry):
        step(2 * k, bufA, bufB)
        step(2 * k + 1, bufB, bufA)
        return carry

    lax.fori_loop(0, NBLK // 2, pair, 0)
    wait_s3(NBLK - 1, rpA, entA, uA)
    pltpu.make_async_copy(scA, out_hbm.at[pl.ds(base0, NB)], semo).wait()
    pltpu.make_async_copy(scB, out_hbm.at[pl.ds(base0, NB)], semo).wait()


def kernel(path, ent_table, rel_table, W1, b1, W2, b2, e, q, rs, es,
           action_keys, segment_ids):
    H = path[0, 0, 2]
    e = e.astype(jnp.int32)
    q = q.astype(jnp.int32)
    rs = rs.astype(jnp.int32)
    es = es.astype(jnp.int32)
    action_keys = action_keys.astype(jnp.int32)
    segment_ids = segment_ids.astype(jnp.int32)

    ent_pk = _pack(ent_table)
    rel_pk = _pack(rel_table)
    e_pk, q_pk = _eq_gather(ent_pk, rel_pk, e, q)
    eemb = _unpack_cols(e_pk, B)
    qemb = _unpack_cols(q_pk, B)
    rp, u = _mlp(eemb, H, qemb, W1, b1, W2, b2, rel_table)
    u_pk = _pack(u)
    return _score(rp.reshape(-1), u_pk, ent_pk, rs, es,
                  action_keys, segment_ids)


# speedup vs baseline: 4.0329x; 4.0329x over previous
import functools

import jax
import jax.numpy as jnp
from jax import lax
from jax.experimental import pallas as pl
from jax.experimental.pallas import tpu as pltpu
from jax.experimental.pallas import tpu_sc as plsc

B = 4096
N_ACT = 131072
N_KEYS = 262144
E_VOCAB = 100000
R_VOCAB = 1000
ENT_DIM = 200
REL_DIM = 200
HIST_DIM = 200
NPAIR = ENT_DIM // 2
PK = 128

NC = 2
NS = 16
NW = NC * NS
L = 16


def _mesh():
    return plsc.VectorSubcoreMesh(
        core_axis_name="c", subcore_axis_name="s",
        num_cores=NC, num_subcores=NS)


def _wid():
    return lax.axis_index("s") * NC + lax.axis_index("c")


def _pack(table):
    tu = jax.lax.bitcast_convert_type(table, jnp.uint32)
    lo = (tu[:, :NPAIR] + 0x8000) >> 16
    hi = (tu[:, NPAIR:] + 0x8000) & jnp.uint32(0xFFFF0000)
    pk = jax.lax.bitcast_convert_type(hi | lo, jnp.int32)
    return jnp.pad(pk, ((0, 0), (0, PK - NPAIR)))


def _unpack_cols(pk, n_rows):
    w = jax.lax.bitcast_convert_type(pk[:, :NPAIR], jnp.uint32)
    lo = jax.lax.bitcast_convert_type(w << 16, jnp.float32)
    hi = jax.lax.bitcast_convert_type(w & jnp.uint32(0xFFFF0000), jnp.float32)
    return jnp.concatenate([lo, hi], axis=1)


ROWS_PER_W = B // NW


@functools.partial(
    pl.kernel,
    out_type=(jax.ShapeDtypeStruct((B, PK), jnp.int32),
              jax.ShapeDtypeStruct((B, PK), jnp.int32)),
    mesh=_mesh(),
    scratch_types=[
        pltpu.VMEM((ROWS_PER_W,), jnp.int32),
        pltpu.VMEM((ROWS_PER_W,), jnp.int32),
        pltpu.VMEM((ROWS_PER_W, PK), jnp.int32),
        pltpu.VMEM((ROWS_PER_W, PK), jnp.int32),
        pltpu.SemaphoreType.DMA,
    ],
    compiler_params=pltpu.CompilerParams(use_tc_tiling_on_sc=True),
)
def _eq_gather(ent_hbm, rel_hbm, e_hbm, q_hbm, eout, qout,
               ei_v, qi_v, e_v, q_v, sem):
    base = _wid() * ROWS_PER_W
    pltpu.sync_copy(e_hbm.at[pl.ds(base, ROWS_PER_W)], ei_v)
    pltpu.sync_copy(q_hbm.at[pl.ds(base, ROWS_PER_W)], qi_v)
    c1 = pltpu.async_copy(ent_hbm.at[ei_v], e_v, sem)
    c2 = pltpu.async_copy(rel_hbm.at[qi_v], q_v, sem)
    c1.wait()
    c2.wait()
    pltpu.sync_copy(e_v, eout.at[pl.ds(base, ROWS_PER_W)])
    pltpu.sync_copy(q_v, qout.at[pl.ds(base, ROWS_PER_W)])


BLK = 256


def _mlp_body(e_ref, h_ref, q_ref, w1_ref, b1_ref, w2a_ref, w2b_ref,
              b2a_ref, b2b_ref, rel_ref, rp_ref, u_ref):
    x = jnp.dot(e_ref[...], w1_ref[0:ENT_DIM, :],
                preferred_element_type=jnp.float32)
    x = x + jnp.dot(h_ref[...], w1_ref[ENT_DIM:ENT_DIM + HIST_DIM, :],
                    preferred_element_type=jnp.float32)
    x = x + jnp.dot(q_ref[...], w1_ref[ENT_DIM + HIST_DIM:, :],
                    preferred_element_type=jnp.float32)
    x = jnp.maximum(x + b1_ref[...], 0.0)
    x2a = jnp.dot(x, w2a_ref[...], preferred_element_type=jnp.float32)
    x2a = x2a + b2a_ref[...]
    x2b = jnp.dot(x, w2b_ref[...], preferred_element_type=jnp.float32)
    x2b = x2b + b2b_ref[...]
    rp_ref[...] = lax.dot_general(
        x2a, rel_ref[...], (((1,), (1,)), ((), ())),
        preferred_element_type=jnp.float32)
    u_ref[...] = x2b


def _mlp(eemb, h, qemb, W1, b1, W2, b2, rel_table):
    W2a, W2b = W2[:, :REL_DIM], W2[:, REL_DIM:]
    b2a, b2b = b2[:REL_DIM].reshape(1, -1), b2[REL_DIM:].reshape(1, -1)
    return pl.pallas_call(
        _mlp_body,
        grid=(B // BLK,),
        in_specs=[
            pl.BlockSpec((BLK, ENT_DIM), lambda i: (i, 0)),
            pl.BlockSpec((BLK, HIST_DIM), lambda i: (i, 0)),
            pl.BlockSpec((BLK, REL_DIM), lambda i: (i, 0)),
            pl.BlockSpec((ENT_DIM + HIST_DIM + REL_DIM, 400), lambda i: (0, 0)),
            pl.BlockSpec((1, 400), lambda i: (0, 0)),
            pl.BlockSpec((400, REL_DIM), lambda i: (0, 0)),
            pl.BlockSpec((400, ENT_DIM), lambda i: (0, 0)),
            pl.BlockSpec((1, REL_DIM), lambda i: (0, 0)),
            pl.BlockSpec((1, ENT_DIM), lambda i: (0, 0)),
            pl.BlockSpec((R_VOCAB, REL_DIM), lambda i: (0, 0)),
        ],
        out_specs=[
            pl.BlockSpec((BLK, R_VOCAB), lambda i: (i, 0)),
            pl.BlockSpec((BLK, ENT_DIM), lambda i: (i, 0)),
        ],
        out_shape=[
            jax.ShapeDtypeStruct((B, R_VOCAB), jnp.float32),
            jax.ShapeDtypeStruct((B, ENT_DIM), jnp.float32),
        ],
    )(eemb, h, qemb, W1, b1.reshape(1, -1), W2a, W2b, b2a, b2b, rel_table)


KPW = N_KEYS // NW
NB = 128
NBLK = KPW // NB
NG = NB // L
MAXR = 32
ROT = 9
UNR = 10


@functools.partial(
    pl.kernel,
    out_type=jax.ShapeDtypeStruct((N_KEYS,), jnp.float32),
    mesh=_mesh(),
    scratch_types=[
        pltpu.VMEM((KPW,), jnp.int32),
        pltpu.VMEM((KPW,), jnp.int32),
        pltpu.VMEM((NB,), jnp.int32),
        pltpu.VMEM((NB,), jnp.int32),
        pltpu.VMEM((NB,), jnp.int32),
        pltpu.VMEM((NB,), jnp.float32),
        pltpu.VMEM((NB,), jnp.float32),
        pltpu.VMEM((NB, PK), jnp.int32),
        pltpu.VMEM((NB, PK), jnp.int32),
        pltpu.VMEM((NB, PK), jnp.int32),
        pltpu.VMEM((NB, PK), jnp.int32),
        pltpu.VMEM((NB,), jnp.float32),
        pltpu.VMEM((NB,), jnp.float32),
        pltpu.SemaphoreType.DMA,
        pltpu.SemaphoreType.DMA,
        pltpu.SemaphoreType.DMA,
    ],
    compiler_params=pltpu.CompilerParams(use_tc_tiling_on_sc=True, needs_layout_passes=False),
)
def _score(rp_hbm, u_hbm, ent_hbm, rs_hbm, es_hbm, ak_hbm, seg_hbm, out_hbm,
           ak_v, seg_v, rs_v, es_v, rpi_v, rpA, rpB, entA, entB, uA, uB,
           scA, scB, sem2, sem3, semo):
    base0 = _wid() * KPW

    def issue_s2(bi):
        idx = ak_v.at[pl.ds(bi * NB, NB)]
        pltpu.async_copy(rs_hbm.at[idx], rs_v, sem2)
        pltpu.async_copy(es_hbm.at[idx], es_v, sem2)

    def wait_s2():
        pltpu.make_async_copy(rs_hbm.at[ak_v.at[pl.ds(0, NB)]], rs_v, sem2).wait()
        pltpu.make_async_copy(es_hbm.at[ak_v.at[pl.ds(0, NB)]], es_v, sem2).wait()

    def block_meta(bi):
        seg0 = seg_v[pl.ds(bi * NB, L)][0]
        segl = seg_v[pl.ds(bi * NB + NB - L, L)][L - 1]
        base_u = jnp.minimum(seg0 & ~7, B - MAXR)
        is_lin = (segl - base_u) < MAXR
        return is_lin, base_u

    def issue_s3(bi, rp_b, ent_b, u_b):
        for g in range(NG):
            s16 = seg_v[pl.ds(bi * NB + g * L, L)]
            r16 = rs_v[pl.ds(g * L, L)]
            rpi_v[pl.ds(g * L, L)] = s16 * R_VOCAB + r16
        pltpu.async_copy(rp_hbm.at[rpi_v], rp_b, sem3)
        pltpu.async_copy(ent_hbm.at[es_v], ent_b, sem3)
        is_lin, base_u = block_meta(bi)

        @pl.when(is_lin)
        def _():
            pltpu.async_copy(u_hbm.at[pl.ds(pl.multiple_of(base_u, 8), MAXR)],
                             u_b.at[pl.ds(0, MAXR)], sem3)

        @pl.when(jnp.logical_not(is_lin))
        def _():
            pltpu.async_copy(u_hbm.at[seg_v.at[pl.ds(bi * NB, NB)]], u_b, sem3)

    def wait_s3(bi, rp_b, ent_b, u_b):
        pltpu.make_async_copy(rp_hbm.at[rpi_v], rp_b, sem3).wait()
        pltpu.make_async_copy(ent_hbm.at[es_v], ent_b, sem3).wait()
        is_lin, base_u = block_meta(bi)

        @pl.when(is_lin)
        def _():
            pltpu.make_async_copy(u_hbm.at[pl.ds(pl.multiple_of(base_u, 8), MAXR)],
                                  u_b.at[pl.ds(0, MAXR)], sem3).wait()

        @pl.when(jnp.logical_not(is_lin))
        def _():
            pltpu.make_async_copy(u_hbm.at[seg_v.at[pl.ds(0, NB)]],
                                  u_b, sem3).wait()

    def compute(bi, rp_b, ent_b, u_b, sc_b):
        rows = [lax.iota(jnp.int32, L) + g * L for g in range(NG)]
        zero = jnp.zeros((L,), jnp.float32)
        is_lin, base_u = block_meta(bi)
        m = is_lin.astype(jnp.int32)
        urows = []
        for g in range(NG):
            s16 = seg_v[pl.ds(bi * NB + g * L, L)]
            urows.append(m * (s16 - base_u) + (1 - m) * rows[g])

        col0 = (lax.iota(jnp.int32, L) * ROT) % NPAIR

        def dstep(j, accs):
            base_c = col0 + j * UNR
            cols = []
            for t in range(UNR):
                c = base_c + t
                cols.append(jnp.where(c >= NPAIR, c - NPAIR, c))
            out = []
            for g in range(NG):
                a = accs[g]
                for t in range(UNR):
                    epk = plsc.load_gather(ent_b, [rows[g], cols[t]])
                    upk = plsc.load_gather(u_b, [urows[g], cols[t]])
                    e0, e1 = plsc.unpack(plsc.bitcast(epk, jnp.bfloat16),
                                         format=plsc.PackFormat.INTERLEAVED)
                    u0, u1 = plsc.unpack(plsc.bitcast(upk, jnp.bfloat16),
                                         format=plsc.PackFormat.INTERLEAVED)
                    a = a + e0 * u0 + e1 * u1
                out.append(a)
            return tuple(out)

        accs = lax.fori_loop(0, NPAIR // UNR, dstep, (zero,) * NG)
        for g in range(NG):
            sc_b[pl.ds(g * L, L)] = accs[g] + rp_b[pl.ds(g * L, L)]

    def step(bi, cur, nxt):
        rp_c, ent_c, u_c, sc_c = cur
        rp_n, ent_n, u_n, _ = nxt
        base = base0 + bi * NB
        bnext = lax.min(bi + 1, NBLK - 1)
        wait_s3(bi, rp_c, ent_c, u_c)
        issue_s2(bnext)
        @pl.when(bi >= 2)
        def _():
            pltpu.make_async_copy(sc_c, out_hbm.at[pl.ds(base, NB)], semo).wait()
        compute(bi, rp_c, ent_c, u_c, sc_c)
        pltpu.async_copy(sc_c, out_hbm.at[pl.ds(base, NB)], semo)
        wait_s2()
        issue_s3(bnext, rp_n, ent_n, u_n)

    bufA = (rpA, entA, uA, scA)
    bufB = (rpB, entB, uB, scB)

    pltpu.sync_copy(ak_hbm.at[pl.ds(base0, KPW)], ak_v)
    pltpu.sync_copy(seg_hbm.at[pl.ds(base0, KPW)], seg_v)
    issue_s2(0)
    wait_s2()
    issue_s3(0, rpA, entA, uA)

    def pair(k, carry):
        step(2 * k, bufA, bufB)
        step(2 * k + 1, bufB, bufA)
        return carry

    lax.fori_loop(0, NBLK // 2, pair, 0)
    wait_s3(NBLK - 1, rpA, entA, uA)
    pltpu.make_async_copy(scA, out_hbm.at[pl.ds(base0, NB)], semo).wait()
    pltpu.make_async_copy(scB, out_hbm.at[pl.ds(base0, NB)], semo).wait()


def kernel(path, ent_table, rel_table, W1, b1, W2, b2, e, q, rs, es,
           action_keys, segment_ids):
    H = path[0, 0, 2]
    e = e.astype(jnp.int32)
    q = q.astype(jnp.int32)
    rs = rs.astype(jnp.int32)
    es = es.astype(jnp.int32)
    action_keys = action_keys.astype(jnp.int32)
    segment_ids = segment_ids.astype(jnp.int32)

    ent_pk = _pack(ent_table)
    rel_pk = _pack(rel_table)
    e_pk, q_pk = _eq_gather(ent_pk, rel_pk, e, q)
    eemb = _unpack_cols(e_pk, B)
    qemb = _unpack_cols(q_pk, B)
    rp, u = _mlp(eemb, H, qemb, W1, b1, W2, b2, rel_table)
    u_pk = _pack(u)
    return _score(rp.reshape(-1), u_pk, ent_pk, rs, es,
                  action_keys, segment_ids)
